# baseline (speedup 1.0000x reference)
import jax
import jax.numpy as jnp
from jax.experimental import pallas as pl
from jax.experimental.pallas import tpu as pltpu

_HIDDEN = 4096
_EXPERTS = 64
_BT = 1024
_NEG = -3.0e38


def _gating_body(x_ref, w_ref, wout_ref, iout_ref):
    lg = jax.lax.dot_general(
        w_ref[...], x_ref[...],
        (((1,), (1,)), ((), ())),
        preferred_element_type=jnp.float32,
    )
    eid = jax.lax.broadcasted_iota(jnp.int32, lg.shape, 0)
    m1 = jnp.max(lg, axis=0, keepdims=True)
    i1 = jnp.min(jnp.where(lg == m1, eid, _EXPERTS), axis=0, keepdims=True)
    masked = jnp.where(eid == i1, _NEG, lg)
    m2 = jnp.max(masked, axis=0, keepdims=True)
    i2 = jnp.min(jnp.where(masked == m2, eid, _EXPERTS), axis=0, keepdims=True)
    e2 = jnp.exp(m2 - m1)
    d = 1.0 + e2
    w1 = 1.0 / d
    w2 = e2 / d
    wout_ref[...] = jnp.concatenate([w1, w2], axis=0).T
    iout_ref[...] = jnp.concatenate([i1, i2], axis=0).T


def kernel(x, W, top_k):
    b, s, h = x.shape
    tokens = b * s
    x2 = x.reshape(tokens, h)
    wout, iout = pl.pallas_call(
        _gating_body,
        grid=(tokens // _BT,),
        in_specs=[
            pl.BlockSpec((_BT, h), lambda i: (i, 0)),
            pl.BlockSpec((_EXPERTS, h), lambda i: (0, 0)),
        ],
        out_specs=[
            pl.BlockSpec((_BT, 2), lambda i: (i, 0)),
            pl.BlockSpec((_BT, 2), lambda i: (i, 0)),
        ],
        out_shape=[
            jax.ShapeDtypeStruct((tokens, 2), jnp.float32),
            jax.ShapeDtypeStruct((tokens, 2), jnp.int32),
        ],
        compiler_params=pltpu.CompilerParams(
            dimension_semantics=("parallel",),
        ),
    )(x2, W)
    return wout.reshape(b, s, 2), iout.reshape(b, s, 2)


# speedup vs baseline: 1.2029x; 1.2029x over previous
import jax
import jax.numpy as jnp
from jax.experimental import pallas as pl
from jax.experimental.pallas import tpu as pltpu

_HIDDEN = 4096
_EXPERTS = 64
_BT = 1024
_NEG = -3.0e38
_H2 = _HIDDEN // 2


def _gating_body(xlo_ref, xhi_ref, wlo_ref, whi_ref, wout_ref, iout_ref):
    dn = (((1,), (1,)), ((), ()))
    lg = jax.lax.dot_general(
        wlo_ref[...], xlo_ref[...], dn, preferred_element_type=jnp.float32,
    ) + jax.lax.dot_general(
        whi_ref[...], xhi_ref[...], dn, preferred_element_type=jnp.float32,
    )
    eid = jax.lax.broadcasted_iota(jnp.int32, lg.shape, 0)
    m1 = jnp.max(lg, axis=0, keepdims=True)
    i1 = jnp.min(jnp.where(lg == m1, eid, _EXPERTS), axis=0, keepdims=True)
    masked = jnp.where(eid == i1, _NEG, lg)
    m2 = jnp.max(masked, axis=0, keepdims=True)
    i2 = jnp.min(jnp.where(masked == m2, eid, _EXPERTS), axis=0, keepdims=True)
    e2 = jnp.exp(m2 - m1)
    d = 1.0 + e2
    w1 = 1.0 / d
    w2 = e2 / d
    wout_ref[...] = jnp.concatenate([w1, w2], axis=0)
    iout_ref[...] = jnp.concatenate([i1, i2], axis=0)


def kernel(x, W, top_k):
    b, s, h = x.shape
    tokens = b * s
    x2 = x.reshape(tokens, h)
    wout, iout = pl.pallas_call(
        _gating_body,
        grid=(tokens // _BT,),
        in_specs=[
            pl.BlockSpec((_BT, _H2), lambda i: (i, 0)),
            pl.BlockSpec((_BT, _H2), lambda i: (i, 1)),
            pl.BlockSpec((_EXPERTS, _H2), lambda i: (0, 0)),
            pl.BlockSpec((_EXPERTS, _H2), lambda i: (0, 1)),
        ],
        out_specs=[
            pl.BlockSpec((2, _BT), lambda i: (0, i)),
            pl.BlockSpec((2, _BT), lambda i: (0, i)),
        ],
        out_shape=[
            jax.ShapeDtypeStruct((2, tokens), jnp.float32),
            jax.ShapeDtypeStruct((2, tokens), jnp.int32),
        ],
        compiler_params=pltpu.CompilerParams(
            dimension_semantics=("parallel",),
        ),
    )(x2, x2, W, W)
    wt = wout.T.reshape(b, s, 2)
    it = iout.T.reshape(b, s, 2)
    return wt, it


# speedup vs baseline: 1.2051x; 1.0018x over previous
import jax
import jax.numpy as jnp
from jax.experimental import pallas as pl
from jax.experimental.pallas import tpu as pltpu

_HIDDEN = 4096
_EXPERTS = 64
_BT = 1024
_NEG = -3.0e38


def _gating_body(x_ref, w_ref, wout_ref, iout_ref):
    lg = jax.lax.dot_general(
        w_ref[...], x_ref[...],
        (((1,), (1,)), ((), ())),
        preferred_element_type=jnp.float32,
    )
    eid = jax.lax.broadcasted_iota(jnp.int32, lg.shape, 0)
    m1 = jnp.max(lg, axis=0, keepdims=True)
    i1 = jnp.min(jnp.where(lg == m1, eid, _EXPERTS), axis=0, keepdims=True)
    masked = jnp.where(eid == i1, _NEG, lg)
    m2 = jnp.max(masked, axis=0, keepdims=True)
    i2 = jnp.min(jnp.where(masked == m2, eid, _EXPERTS), axis=0, keepdims=True)
    e2 = jnp.exp(m2 - m1)
    d = 1.0 + e2
    w1 = 1.0 / d
    w2 = e2 / d
    wout_ref[...] = jnp.concatenate([w1, w2], axis=0)
    iout_ref[...] = jnp.concatenate([i1, i2], axis=0)


def kernel(x, W, top_k):
    b, s, h = x.shape
    tokens = b * s
    x2 = x.reshape(tokens, h)
    wout, iout = pl.pallas_call(
        _gating_body,
        grid=(tokens // _BT,),
        in_specs=[
            pl.BlockSpec((_BT, h), lambda i: (i, 0)),
            pl.BlockSpec((_EXPERTS, h), lambda i: (0, 0)),
        ],
        out_specs=[
            pl.BlockSpec((2, _BT), lambda i: (0, i)),
            pl.BlockSpec((2, _BT), lambda i: (0, i)),
        ],
        out_shape=[
            jax.ShapeDtypeStruct((2, tokens), jnp.float32),
            jax.ShapeDtypeStruct((2, tokens), jnp.int32),
        ],
        compiler_params=pltpu.CompilerParams(
            dimension_semantics=("parallel",),
        ),
    )(x2, W)
    wt = wout.T.reshape(b, s, 2)
    it = iout.T.reshape(b, s, 2)
    return wt, it
